# baseline (speedup 1.0000x reference)
import jax
import jax.numpy as jnp
from jax import lax
from jax.experimental import pallas as pl
from jax.experimental.pallas import tpu as pltpu
from jax.experimental.pallas import tpu_sc as plsc

N_NODES = 10000
N_EDGES = 320000
IN_CH = 128
OUT_CH = 64

N_PAD = 10240
CHUNK = N_PAD // 16
HALF = CHUNK // 2
EW = 128
NBLK = 12
NOUT = 13
NWIN = NBLK * NOUT
EROWS = N_EDGES // EW


def _mm_body(x_ref, w_ref, out_ref):
    h = jnp.dot(x_ref[...], w_ref[...], preferred_element_type=jnp.float32)
    out_ref[0] = h[:, :OUT_CH]
    out_ref[1] = h[:, OUT_CH:]


def _matmul(x, wcat):
    blk = 2048
    return pl.pallas_call(
        _mm_body,
        grid=(N_PAD // blk,),
        in_specs=[
            pl.BlockSpec((blk, IN_CH), lambda g: (g, 0)),
            pl.BlockSpec((IN_CH, 2 * OUT_CH), lambda g: (0, 0)),
        ],
        out_specs=pl.BlockSpec((2, blk, OUT_CH), lambda g: (0, g, 0)),
        out_shape=jax.ShapeDtypeStruct((2, N_PAD, OUT_CH), jnp.float32),
    )(x, wcat)


def _sc_body(h_pair, src_hbm, dst_hbm, bias_pair, out_mu, out_ls,
             hs_shared, acc_shared, deg_shared,
             h_v, src_v, dst_v, rows_a, rows_b, deg_v, dinv_v, ones_v, bias_v,
             gsem, ssem):
    c = lax.axis_index("c")
    t = lax.axis_index("s")
    row0 = t * CHUNK

    h_pre = pltpu.async_copy(h_pair.at[c].at[pl.ds(row0, HALF)], h_v, gsem)

    def _fill(i, _):
        ones_v[pl.ds(i * 16, 16)] = jnp.ones((16,), jnp.float32)
        return 0
    lax.fori_loop(0, EW // 16, _fill, 0)

    def _dinit(i, _):
        pltpu.sync_copy(ones_v, deg_shared.at[pl.ds(row0 + i * EW, EW)])
        return 0
    lax.fori_loop(0, CHUNK // EW, _dinit, 0)
    plsc.subcore_barrier()

    erow0 = t * NWIN

    def _hist_blk(ob, _):
        pltpu.sync_copy(dst_hbm.at[pl.ds(erow0 + ob * NBLK, NBLK)], dst_v)

        def _fire(j, _):
            pltpu.async_copy(ones_v, deg_shared.at[dst_v.at[j]], ssem,
                             add=True)
            return 0
        lax.fori_loop(0, NBLK, _fire, 0)

        def _drain(j, _):
            pltpu.make_async_copy(ones_v, deg_shared.at[dst_v.at[j]],
                                  ssem).wait()
            return 0
        lax.fori_loop(0, NBLK, _drain, 0)
        return 0
    lax.fori_loop(0, NOUT, _hist_blk, 0)

    @pl.when(t < EROWS - 16 * NWIN)
    def _():
        pltpu.sync_copy(dst_hbm.at[pl.ds(16 * NWIN + t, 1)],
                        dst_v.at[pl.ds(0, 1)])
        pltpu.sync_copy(ones_v, deg_shared.at[dst_v.at[0]], add=True)
    plsc.subcore_barrier()

    pltpu.sync_copy(deg_shared.at[pl.ds(row0, CHUNK)], deg_v)

    def _rsqrt(k, _):
        d = deg_v[pl.ds(k * 16, 16)]
        hd = 0.5 * d
        y = jnp.full((16,), 0.0009765625, jnp.float32)
        for _i in range(26):
            y = y * (1.5 - hd * y * y)
        dinv_v[pl.ds(k * 16, 16)] = y
        return 0
    lax.fori_loop(0, CHUNK // 16, _rsqrt, 0)

    for half in range(2):
        r0 = row0 + half * HALF
        if half == 0:
            h_pre.wait()
        else:
            pltpu.sync_copy(h_pair.at[c].at[pl.ds(r0, HALF)], h_v)

        def _scale(i, _):
            s = plsc.load_gather(
                dinv_v, [jnp.broadcast_to(half * HALF + i, (16,))])
            for k in range(OUT_CH // 16):
                h_v[i, pl.ds(k * 16, 16)] = h_v[i, pl.ds(k * 16, 16)] * s
            return 0
        lax.fori_loop(0, HALF, _scale, 0)
        pltpu.sync_copy(h_v, hs_shared.at[pl.ds(r0, HALF)])
        pltpu.sync_copy(h_v, acc_shared.at[pl.ds(r0, HALF)])
    plsc.subcore_barrier()

    hs_c = hs_shared

    def _edge_blk(ob, _):
        pltpu.sync_copy(src_hbm.at[pl.ds(erow0 + ob * NBLK, NBLK)], src_v)
        pltpu.sync_copy(dst_hbm.at[pl.ds(erow0 + ob * NBLK, NBLK)], dst_v)
        pltpu.async_copy(hs_c.at[src_v.at[0]], rows_a, gsem)

        def _pair(jj, _):
            j0 = 2 * jj
            j1 = j0 + 1
            pltpu.make_async_copy(hs_c.at[src_v.at[j0]], rows_a, gsem).wait()
            pltpu.async_copy(rows_a, acc_shared.at[dst_v.at[j0]], ssem,
                             add=True)

            @pl.when(jj > 0)
            def _():
                pltpu.make_async_copy(rows_b, acc_shared.at[dst_v.at[j0 - 1]],
                                      ssem).wait()
            pltpu.async_copy(hs_c.at[src_v.at[j1]], rows_b, gsem)
            pltpu.make_async_copy(hs_c.at[src_v.at[j1]], rows_b, gsem).wait()
            pltpu.make_async_copy(rows_a, acc_shared.at[dst_v.at[j0]],
                                  ssem).wait()

            @pl.when(jj < NBLK // 2 - 1)
            def _():
                pltpu.async_copy(hs_c.at[src_v.at[j0 + 2]], rows_a, gsem)
            pltpu.async_copy(rows_b, acc_shared.at[dst_v.at[j1]], ssem,
                             add=True)
            return 0
        lax.fori_loop(0, NBLK // 2, _pair, 0)
        pltpu.make_async_copy(rows_b, acc_shared.at[dst_v.at[NBLK - 1]],
                              ssem).wait()
        return 0
    lax.fori_loop(0, NOUT, _edge_blk, 0)

    @pl.when(t < EROWS - 16 * NWIN)
    def _():
        pltpu.sync_copy(src_hbm.at[pl.ds(16 * NWIN + t, 1)],
                        src_v.at[pl.ds(0, 1)])
        pltpu.sync_copy(dst_hbm.at[pl.ds(16 * NWIN + t, 1)],
                        dst_v.at[pl.ds(0, 1)])
        pltpu.sync_copy(hs_c.at[src_v.at[0]], rows_a)
        pltpu.sync_copy(rows_a, acc_shared.at[dst_v.at[0]], add=True)
    plsc.subcore_barrier()

    pltpu.sync_copy(bias_pair.at[c], bias_v)
    bvs = [bias_v[pl.ds(k * 16, 16)] for k in range(OUT_CH // 16)]
    tail = N_NODES - 15 * CHUNK - HALF
    for half in range(2):
        r0 = row0 + half * HALF
        pltpu.sync_copy(acc_shared.at[pl.ds(r0, HALF)], h_v)

        def _final(i, _):
            s = plsc.load_gather(
                dinv_v, [jnp.broadcast_to(half * HALF + i, (16,))])
            for k in range(OUT_CH // 16):
                h_v[i, pl.ds(k * 16, 16)] = (
                    h_v[i, pl.ds(k * 16, 16)] * s + bvs[k])
            return 0
        lax.fori_loop(0, HALF, _final, 0)
        for cc, out_ref in ((0, out_mu), (1, out_ls)):
            if half == 0:
                @pl.when(c == cc)
                def _(out_ref=out_ref, r0=r0):
                    pltpu.sync_copy(h_v, out_ref.at[pl.ds(r0, HALF)])
            else:
                @pl.when((c == cc) & (t < 15))
                def _(out_ref=out_ref, r0=r0):
                    pltpu.sync_copy(h_v, out_ref.at[pl.ds(r0, HALF)])

                @pl.when((c == cc) & (t == 15))
                def _(out_ref=out_ref):
                    pltpu.sync_copy(
                        h_v.at[pl.ds(0, tail)],
                        out_ref.at[pl.ds(N_NODES - tail, tail)])


_sc_call = pl.kernel(
    _sc_body,
    out_type=(jax.ShapeDtypeStruct((N_NODES, OUT_CH), jnp.float32),
              jax.ShapeDtypeStruct((N_NODES, OUT_CH), jnp.float32)),
    mesh=plsc.VectorSubcoreMesh(core_axis_name="c", subcore_axis_name="s"),
    compiler_params=pltpu.CompilerParams(needs_layout_passes=False,
                                         use_tc_tiling_on_sc=False),
    scratch_types=[
        pltpu.VMEM_SHARED((N_PAD, OUT_CH), jnp.float32),
        pltpu.VMEM_SHARED((N_PAD, OUT_CH), jnp.float32),
        pltpu.VMEM_SHARED((N_PAD,), jnp.float32),
        pltpu.VMEM((HALF, OUT_CH), jnp.float32),
        pltpu.VMEM((NBLK, EW), jnp.int32),
        pltpu.VMEM((NBLK, EW), jnp.int32),
        pltpu.VMEM((EW, OUT_CH), jnp.float32),
        pltpu.VMEM((EW, OUT_CH), jnp.float32),
        pltpu.VMEM((CHUNK,), jnp.float32),
        pltpu.VMEM((CHUNK,), jnp.float32),
        pltpu.VMEM((EW,), jnp.float32),
        pltpu.VMEM((OUT_CH,), jnp.float32),
        pltpu.SemaphoreType.DMA,
        pltpu.SemaphoreType.DMA,
    ],
)


@jax.jit
def kernel(x, edge_index, W_mu, b_mu, W_logstd, b_logstd):
    wcat = jnp.concatenate([W_mu, W_logstd], axis=1)
    h_pair = _matmul(x, wcat)

    src_p = edge_index[0].astype(jnp.int32).reshape(EROWS, EW)
    dst_p = edge_index[1].astype(jnp.int32).reshape(EROWS, EW)
    bias_pair = jnp.stack([b_mu, b_logstd])

    out_mu, out_ls = _sc_call(h_pair, src_p, dst_p, bias_pair)
    return out_mu, out_ls


# speedup vs baseline: 1.0637x; 1.0637x over previous
import jax
import jax.numpy as jnp
from jax import lax
from jax.experimental import pallas as pl
from jax.experimental.pallas import tpu as pltpu
from jax.experimental.pallas import tpu_sc as plsc

N_NODES = 10000
N_EDGES = 320000
IN_CH = 128
OUT_CH = 64

N_PAD = 10240
CHUNK = N_PAD // 16
SUB = CHUNK // 8
EW = 128
NBLK = 24
NFULL = 6
TAILW = 12
NWIN = NBLK * NFULL + TAILW
EROWS = N_EDGES // EW


def _mm_body(x_ref, w_ref, out_ref):
    h = jnp.dot(x_ref[...], w_ref[...], preferred_element_type=jnp.float32)
    out_ref[0] = h[:, :OUT_CH]
    out_ref[1] = h[:, OUT_CH:]


def _matmul(x, wcat):
    blk = 2048
    return pl.pallas_call(
        _mm_body,
        grid=(N_PAD // blk,),
        in_specs=[
            pl.BlockSpec((blk, IN_CH), lambda g: (g, 0)),
            pl.BlockSpec((IN_CH, 2 * OUT_CH), lambda g: (0, 0)),
        ],
        out_specs=pl.BlockSpec((2, blk, OUT_CH), lambda g: (0, g, 0)),
        out_shape=jax.ShapeDtypeStruct((2, N_PAD, OUT_CH), jnp.float32),
    )(x, wcat)


def _sc_body(h_pair, src_hbm, dst_hbm, bias_pair, out_mu, out_ls,
             hs_shared, acc_shared, deg_shared,
             h_v, src_v, dst_v, rows_a, rows_b, rows_c, rows_d,
             deg_v, dinv_v, ones_v, bias_v, gsem, ssem):
    c = lax.axis_index("c")
    t = lax.axis_index("s")
    row0 = t * CHUNK
    erow0 = t * NWIN
    n_extra = EROWS - 16 * NWIN
    bufs = (rows_a, rows_b, rows_c, rows_d)

    h_pre = pltpu.async_copy(h_pair.at[c].at[pl.ds(row0, SUB)], h_v, gsem)

    def _fill(i, _):
        ones_v[pl.ds(i * 16, 16)] = jnp.ones((16,), jnp.float32)
        return 0
    lax.fori_loop(0, EW // 16, _fill, 0)

    def _dinit(i, _):
        pltpu.sync_copy(ones_v, deg_shared.at[pl.ds(row0 + i * EW, EW)])
        return 0
    lax.fori_loop(0, CHUNK // EW, _dinit, 0)
    plsc.subcore_barrier()

    def _hist_block(ob, nw):
        pltpu.sync_copy(dst_hbm.at[pl.ds(erow0 + ob * NBLK, nw)],
                        dst_v.at[pl.ds(0, nw)])

        def _fire(j, _):
            pltpu.async_copy(ones_v, deg_shared.at[dst_v.at[j]], ssem,
                             add=True)
            return 0
        lax.fori_loop(0, nw, _fire, 0)

        def _drain(j, _):
            pltpu.make_async_copy(ones_v, deg_shared.at[dst_v.at[j]],
                                  ssem).wait()
            return 0
        lax.fori_loop(0, nw, _drain, 0)
        return 0

    lax.fori_loop(0, NFULL, lambda ob, _: _hist_block(ob, NBLK), 0)
    _hist_block(NFULL, TAILW)

    @pl.when(t < n_extra)
    def _():
        pltpu.sync_copy(dst_hbm.at[pl.ds(16 * NWIN + t, 1)],
                        dst_v.at[pl.ds(0, 1)])
        pltpu.sync_copy(ones_v, deg_shared.at[dst_v.at[0]], add=True)
    plsc.subcore_barrier()

    pltpu.sync_copy(deg_shared.at[pl.ds(row0, CHUNK)], deg_v)

    def _rsqrt(k, _):
        d = deg_v[pl.ds(k * 16, 16)]
        hd = 0.5 * d
        y = jnp.full((16,), 0.0009765625, jnp.float32)
        for _i in range(26):
            y = y * (1.5 - hd * y * y)
        dinv_v[pl.ds(k * 16, 16)] = y
        return 0
    lax.fori_loop(0, CHUNK // 16, _rsqrt, 0)

    for q in range(CHUNK // SUB):
        r0 = row0 + q * SUB
        if q == 0:
            h_pre.wait()
        else:
            pltpu.sync_copy(h_pair.at[c].at[pl.ds(r0, SUB)], h_v)

        def _scale(i, _):
            s = plsc.load_gather(
                dinv_v, [jnp.broadcast_to(q * SUB + i, (16,))])
            for k in range(OUT_CH // 16):
                h_v[i, pl.ds(k * 16, 16)] = h_v[i, pl.ds(k * 16, 16)] * s
            return 0
        lax.fori_loop(0, SUB, _scale, 0)
        pltpu.sync_copy(h_v, hs_shared.at[pl.ds(r0, SUB)])
        pltpu.sync_copy(h_v, acc_shared.at[pl.ds(r0, SUB)])
    plsc.subcore_barrier()

    def _edge_block(ob, nw):
        pltpu.sync_copy(src_hbm.at[pl.ds(erow0 + ob * NBLK, nw)],
                        src_v.at[pl.ds(0, nw)])
        pltpu.sync_copy(dst_hbm.at[pl.ds(erow0 + ob * NBLK, nw)],
                        dst_v.at[pl.ds(0, nw)])
        pltpu.async_copy(hs_shared.at[src_v.at[0]], bufs[0], gsem)
        for w in range(nw):
            bw = bufs[w % 4]
            pltpu.make_async_copy(hs_shared.at[src_v.at[w]], bw, gsem).wait()
            pltpu.async_copy(bw, acc_shared.at[dst_v.at[w]], ssem, add=True)
            if w >= 3:
                pltpu.make_async_copy(bufs[(w - 3) % 4],
                                      acc_shared.at[dst_v.at[w - 3]],
                                      ssem).wait()
            if w + 1 < nw:
                pltpu.async_copy(hs_shared.at[src_v.at[w + 1]],
                                 bufs[(w + 1) % 4], gsem)
        for w in range(max(0, nw - 3), nw):
            pltpu.make_async_copy(bufs[w % 4], acc_shared.at[dst_v.at[w]],
                                  ssem).wait()
        return 0

    lax.fori_loop(0, NFULL, lambda ob, _: _edge_block(ob, NBLK), 0)
    _edge_block(NFULL, TAILW)

    @pl.when(t < n_extra)
    def _():
        pltpu.sync_copy(src_hbm.at[pl.ds(16 * NWIN + t, 1)],
                        src_v.at[pl.ds(0, 1)])
        pltpu.sync_copy(dst_hbm.at[pl.ds(16 * NWIN + t, 1)],
                        dst_v.at[pl.ds(0, 1)])
        pltpu.sync_copy(hs_shared.at[src_v.at[0]], rows_a)
        pltpu.sync_copy(rows_a, acc_shared.at[dst_v.at[0]], add=True)
    plsc.subcore_barrier()

    pltpu.sync_copy(bias_pair.at[c], bias_v)
    bvs = [bias_v[pl.ds(k * 16, 16)] for k in range(OUT_CH // 16)]
    n_sub15 = (N_NODES - 15 * CHUNK) // SUB
    for q in range(CHUNK // SUB):
        r0 = row0 + q * SUB
        pltpu.sync_copy(acc_shared.at[pl.ds(r0, SUB)], h_v)

        def _final(i, _):
            s = plsc.load_gather(
                dinv_v, [jnp.broadcast_to(q * SUB + i, (16,))])
            for k in range(OUT_CH // 16):
                h_v[i, pl.ds(k * 16, 16)] = (
                    h_v[i, pl.ds(k * 16, 16)] * s + bvs[k])
            return 0
        lax.fori_loop(0, SUB, _final, 0)
        for cc, out_ref in ((0, out_mu), (1, out_ls)):
            if q < n_sub15:
                @pl.when(c == cc)
                def _(out_ref=out_ref, r0=r0):
                    pltpu.sync_copy(h_v, out_ref.at[pl.ds(r0, SUB)])
            else:
                @pl.when((c == cc) & (t < 15))
                def _(out_ref=out_ref, r0=r0):
                    pltpu.sync_copy(h_v, out_ref.at[pl.ds(r0, SUB)])


_sc_call = pl.kernel(
    _sc_body,
    out_type=(jax.ShapeDtypeStruct((N_NODES, OUT_CH), jnp.float32),
              jax.ShapeDtypeStruct((N_NODES, OUT_CH), jnp.float32)),
    mesh=plsc.VectorSubcoreMesh(core_axis_name="c", subcore_axis_name="s"),
    compiler_params=pltpu.CompilerParams(needs_layout_passes=False,
                                         use_tc_tiling_on_sc=False),
    scratch_types=[
        pltpu.VMEM_SHARED((N_PAD, OUT_CH), jnp.float32),
        pltpu.VMEM_SHARED((N_PAD, OUT_CH), jnp.float32),
        pltpu.VMEM_SHARED((N_PAD,), jnp.float32),
        pltpu.VMEM((SUB, OUT_CH), jnp.float32),
        pltpu.VMEM((NBLK, EW), jnp.int32),
        pltpu.VMEM((NBLK, EW), jnp.int32),
        pltpu.VMEM((EW, OUT_CH), jnp.float32),
        pltpu.VMEM((EW, OUT_CH), jnp.float32),
        pltpu.VMEM((EW, OUT_CH), jnp.float32),
        pltpu.VMEM((EW, OUT_CH), jnp.float32),
        pltpu.VMEM((CHUNK,), jnp.float32),
        pltpu.VMEM((CHUNK,), jnp.float32),
        pltpu.VMEM((EW,), jnp.float32),
        pltpu.VMEM((OUT_CH,), jnp.float32),
        pltpu.SemaphoreType.DMA,
        pltpu.SemaphoreType.DMA,
    ],
)


@jax.jit
def kernel(x, edge_index, W_mu, b_mu, W_logstd, b_logstd):
    wcat = jnp.concatenate([W_mu, W_logstd], axis=1)
    h_pair = _matmul(x, wcat)

    src_p = edge_index[0].astype(jnp.int32).reshape(EROWS, EW)
    dst_p = edge_index[1].astype(jnp.int32).reshape(EROWS, EW)
    bias_pair = jnp.stack([b_mu, b_logstd])

    out_mu, out_ls = _sc_call(h_pair, src_p, dst_p, bias_pair)
    return out_mu, out_ls
